# baseline (speedup 1.0000x reference)
import functools

import jax
import jax.numpy as jnp
from jax.experimental import pallas as pl

_BB = 512


def _body(x3_ref, ec_ref, e0_ref, out_ref):
    d = ec_ref.shape[0]
    bb = out_ref.shape[1]
    mask = jnp.broadcast_to(x3_ref[0] == 0, (d, bb))
    ecb = jnp.broadcast_to(ec_ref[...], (d, bb))
    e0b = jnp.broadcast_to(e0_ref[...], (d, bb))
    out_ref[...] = jnp.where(mask, e0b, ecb)


def kernel(x, embeddings):
    b, t = x.shape
    v, d = embeddings.shape
    x3 = x.T.reshape(t, 1, b).astype(jnp.int32)
    ecol = embeddings[1:t + 1].reshape(-1, 1)
    e0col = embeddings[0].reshape(-1, 1)
    out2 = pl.pallas_call(
        _body,
        grid=(t, b // _BB),
        in_specs=[
            pl.BlockSpec((1, 1, _BB), lambda i, j: (i, 0, j)),
            pl.BlockSpec((d, 1), lambda i, j: (i, 0)),
            pl.BlockSpec((d, 1), lambda i, j: (0, 0)),
        ],
        out_specs=pl.BlockSpec((d, _BB), lambda i, j: (i, j)),
        out_shape=jax.ShapeDtypeStruct((t * d, b), jnp.float32),
    )(x3, ecol, e0col)
    return out2.reshape(t, d, b).transpose(2, 0, 1)
